# Initial kernel scaffold; baseline (speedup 1.0000x reference)
import jax
import jax.numpy as jnp
from jax.experimental import pallas as pl


def kernel(x, edge_index, edge_attr, c1_W1, c1_b1, c1_W2, c1_b2, c2_W1, c2_b1, c2_W2, c2_b2, c3_W1, c3_b1, c3_W2, c3_b2, head_W, head_b):
    raise NotImplementedError("write your pallas kernel here")


# speedup vs baseline: 1.8870x; 1.8870x over previous
import functools

import jax
import jax.numpy as jnp
from jax import lax
from jax.experimental import pallas as pl
from jax.experimental.pallas import tpu as pltpu
from jax.experimental.pallas import tpu_sc as plsc

_N = 10000
_E = 320000
_D = 128
_ED = 16
_H = 32

_NC = 2
_NS = 16
_NW = _NC * _NS
_RPW = 313
_NPAD = _NW * _RPW
_EW = _E // _NW
_C1 = 1000
_CS = 2000
_BATCH = 1024
_ECAP = _E + _BATCH

_mesh = plsc.VectorSubcoreMesh(core_axis_name="c", subcore_axis_name="s")


def _leaky(v):
    return jnp.where(v >= 0, v, 0.01 * v)


@functools.partial(
    pl.kernel,
    out_type=(jax.ShapeDtypeStruct((_NW, _ECAP), jnp.int32),
              jax.ShapeDtypeStruct((_NW, _ECAP), jnp.int32),
              jax.ShapeDtypeStruct((_NW, 16), jnp.int32)),
    mesh=_mesh,
    compiler_params=pltpu.CompilerParams(needs_layout_passes=False),
    scratch_types=[
        pltpu.VMEM((_CS,), jnp.int32),
        pltpu.VMEM((_BATCH + 16,), jnp.int32),
        pltpu.VMEM((_BATCH + 16,), jnp.int32),
        pltpu.VMEM((16,), jnp.int32),
    ],
)
def _bucket(dst_hbm, eid_hbm, dloc_hbm, cnt_hbm, scanb, eidb, dstb, cntv):
    wid = lax.axis_index("s") * _NC + lax.axis_index("c")
    lo = wid * _RPW
    hi = lo + _RPW

    zero16 = jnp.zeros((16,), jnp.int32)

    def initz(k, c):
        eidb[pl.ds(k * 16, 16)] = zero16
        dstb[pl.ds(k * 16, 16)] = zero16
        return c

    lax.fori_loop(0, (_BATCH + 16) // 16, initz, 0)

    def flush(nf):
        pltpu.sync_copy(eidb.at[pl.ds(0, _BATCH)],
                        eid_hbm.at[wid, pl.ds(nf * _BATCH, _BATCH)])
        pltpu.sync_copy(dstb.at[pl.ds(0, _BATCH)],
                        dloc_hbm.at[wid, pl.ds(nf * _BATCH, _BATCH)])

    def scan_chunk(ci, carry):
        off, nf = carry
        cbase = ci * _CS
        pltpu.sync_copy(dst_hbm.at[pl.ds(cbase, _CS)], scanb)

        def step(k, carry):
            off, nf = carry
            d = scanb[pl.ds(k * 16, 16)]
            msk = (d >= lo) & (d < hi)
            eid = lax.iota(jnp.int32, 16) + (cbase + k * 16)
            mi = jnp.where(msk, 1, 0)
            cs = plsc.cumsum(mi)
            pos = off + cs - mi
            plsc.store_scatter(eidb, [pos], eid, mask=msk)
            plsc.store_scatter(dstb, [pos], d - lo, mask=msk)
            off = off + cs[15]
            full = off >= _BATCH

            @pl.when(full)
            def _():
                flush(nf)
                ve = eidb[pl.ds(_BATCH, 16)]
                vd = dstb[pl.ds(_BATCH, 16)]
                eidb[pl.ds(0, 16)] = ve
                dstb[pl.ds(0, 16)] = vd

            off = off - jnp.where(full, _BATCH, 0)
            nf = nf + jnp.where(full, 1, 0)
            return (off, nf)

        return lax.fori_loop(0, _CS // 16, step, (off, nf))

    off, nf = lax.fori_loop(0, _E // _CS, scan_chunk, (0, 0))

    @pl.when(off > 0)
    def _():
        flush(nf)

    cntv[...] = jnp.full((16,), nf * _BATCH + off, jnp.int32)
    pltpu.sync_copy(cntv, cnt_hbm.at[wid])


@functools.partial(
    pl.kernel,
    out_type=jax.ShapeDtypeStruct((_E, _H), jnp.float32),
    mesh=_mesh,
    compiler_params=pltpu.CompilerParams(use_tc_tiling_on_sc=False),
    scratch_types=[
        pltpu.VMEM((_C1,), jnp.int32),
        pltpu.VMEM((_C1,), jnp.int32),
        pltpu.VMEM((_C1, _H), jnp.float32),
        pltpu.VMEM((_C1, _H), jnp.float32),
        pltpu.VMEM((_C1, _H), jnp.float32),
        pltpu.SemaphoreType.DMA,
        pltpu.SemaphoreType.DMA,
    ],
)
def _gather_pre(a_hbm, b_hbm, ep_hbm, src_hbm, dst_hbm, pre_hbm,
                idxs, idxd, bufa, bufb, bufe, sema, semb):
    wid = lax.axis_index("s") * _NC + lax.axis_index("c")

    def chunk(it, carry):
        base = wid * _EW + it * _C1
        pltpu.sync_copy(dst_hbm.at[pl.ds(base, _C1)], idxd)
        pltpu.sync_copy(src_hbm.at[pl.ds(base, _C1)], idxs)
        cpa = pltpu.async_copy(a_hbm.at[idxd], bufa, sema)
        cpb = pltpu.async_copy(b_hbm.at[idxs], bufb, semb)
        pltpu.sync_copy(ep_hbm.at[pl.ds(base, _C1)], bufe)
        cpa.wait()
        cpb.wait()

        def row(r, c2):
            for hh in range(2):
                s = pl.ds(hh * 16, 16)
                bufe[r, s] = bufa[r, s] + bufb[r, s] + bufe[r, s]
            return c2

        lax.fori_loop(0, _C1, row, 0, unroll=2)
        pltpu.sync_copy(bufe, pre_hbm.at[pl.ds(base, _C1)])
        return carry

    lax.fori_loop(0, _EW // _C1, chunk, 0)


@functools.partial(
    pl.kernel,
    out_type=jax.ShapeDtypeStruct((_NPAD, _H), jnp.float32),
    mesh=_mesh,
    compiler_params=pltpu.CompilerParams(use_tc_tiling_on_sc=False),
    scratch_types=[
        pltpu.VMEM((_RPW, _H), jnp.float32),
        pltpu.VMEM((_BATCH,), jnp.int32),
        pltpu.VMEM((_BATCH,), jnp.int32),
        pltpu.VMEM((_BATCH, _H), jnp.float32),
        pltpu.VMEM((16,), jnp.int32),
        pltpu.SemaphoreType.DMA,
    ],
)
def _seg_min(m_hbm, eid_hbm, dloc_hbm, cnt_hbm, out_hbm,
             table, eidg, dstl, mbuf, cntv, sem):
    wid = lax.axis_index("s") * _NC + lax.axis_index("c")
    lo = wid * _RPW
    inf16 = jnp.full((16,), jnp.inf, jnp.float32)

    def initr(r, c):
        table[r, pl.ds(0, 16)] = inf16
        table[r, pl.ds(16, 16)] = inf16
        return c

    lax.fori_loop(0, _RPW, initr, 0)

    pltpu.sync_copy(cnt_hbm.at[wid], cntv)
    cnt = cntv[...][0]
    nb = (cnt + _BATCH - 1) // _BATCH

    def block(fb, c):
        pltpu.sync_copy(eid_hbm.at[wid, pl.ds(fb * _BATCH, _BATCH)], eidg)
        pltpu.sync_copy(dloc_hbm.at[wid, pl.ds(fb * _BATCH, _BATCH)], dstl)
        pltpu.async_copy(m_hbm.at[eidg], mbuf, sem).wait()
        bound = jnp.minimum(cnt - fb * _BATCH, _BATCH)

        def upd(e, c2):
            dl = dstl[pl.ds(e, 16)][0]
            for hh in range(2):
                s = pl.ds(hh * 16, 16)
                table[dl, s] = jnp.minimum(table[dl, s], mbuf[e, s])
            return c2

        lax.fori_loop(0, bound, upd, 0)
        return c

    lax.fori_loop(0, nb, block, 0)

    def finr(r, c):
        for hh in range(2):
            s = pl.ds(hh * 16, 16)
            v = table[r, s]
            v = jnp.where(v == jnp.inf, 0.0, v)
            table[r, s] = jnp.where(v >= 0, v, 0.01 * v)
        return c

    lax.fori_loop(0, _RPW, finr, 0)
    pltpu.sync_copy(table, out_hbm.at[pl.ds(lo, _RPW)])


def _proj_tc(f, wi_t, wj_t, bias):
    nf = f.shape[0]

    def kern(f_ref, wi_ref, wj_ref, b_ref, a_ref, bb_ref):
        xv = f_ref[...]
        a_ref[...] = jnp.dot(xv, wi_ref[...],
                             preferred_element_type=jnp.float32) + b_ref[...]
        bb_ref[...] = jnp.dot(xv, wj_ref[...],
                              preferred_element_type=jnp.float32)

    return pl.pallas_call(
        kern,
        out_shape=(jax.ShapeDtypeStruct((nf, _H), jnp.float32),
                   jax.ShapeDtypeStruct((nf, _H), jnp.float32)),
    )(f, wi_t, wj_t, bias)


def _eproj_tc(ea, w1_t, w2_t, w3_t):
    be = 4000
    grid = _E // be

    def kern(ea_ref, w1_ref, w2_ref, w3_ref, o1_ref, o2_ref, o3_ref):
        a = ea_ref[...]
        o1_ref[...] = jnp.dot(a, w1_ref[...], preferred_element_type=jnp.float32)
        o2_ref[...] = jnp.dot(a, w2_ref[...], preferred_element_type=jnp.float32)
        o3_ref[...] = jnp.dot(a, w3_ref[...], preferred_element_type=jnp.float32)

    wspec = pl.BlockSpec((_ED, _H), lambda i: (0, 0))
    ospec = pl.BlockSpec((be, _H), lambda i: (i, 0))
    return pl.pallas_call(
        kern,
        grid=(grid,),
        in_specs=[pl.BlockSpec((be, _ED), lambda i: (i, 0)), wspec, wspec, wspec],
        out_specs=(ospec, ospec, ospec),
        out_shape=(jax.ShapeDtypeStruct((_E, _H), jnp.float32),) * 3,
    )(ea, w1_t, w2_t, w3_t)


def _mlp2_tc(pre, w2_t, b2):
    be = 4000
    grid = _E // be

    def kern(p_ref, w_ref, b_ref, o_ref):
        hv = _leaky(p_ref[...])
        o_ref[...] = jnp.dot(hv, w_ref[...],
                             preferred_element_type=jnp.float32) + b_ref[...]

    return pl.pallas_call(
        kern,
        grid=(grid,),
        in_specs=[pl.BlockSpec((be, _H), lambda i: (i, 0)),
                  pl.BlockSpec((_H, _H), lambda i: (0, 0)),
                  pl.BlockSpec((1, _H), lambda i: (0, 0))],
        out_specs=pl.BlockSpec((be, _H), lambda i: (i, 0)),
        out_shape=jax.ShapeDtypeStruct((_E, _H), jnp.float32),
    )(pre, w2_t, b2)


def _head_tc(x, h3, wx_t, wh_t, bias):
    def kern(x_ref, h_ref, wx_ref, wh_ref, b_ref, o_ref):
        o_ref[...] = (jnp.dot(x_ref[...], wx_ref[...],
                              preferred_element_type=jnp.float32)
                      + jnp.dot(h_ref[...], wh_ref[...],
                                preferred_element_type=jnp.float32)
                      + b_ref[...])

    return pl.pallas_call(
        kern,
        out_shape=jax.ShapeDtypeStruct((_N, 1), jnp.float32),
    )(x, h3, wx_t, wh_t, bias)


def kernel(x, edge_index, edge_attr, c1_W1, c1_b1, c1_W2, c1_b2,
           c2_W1, c2_b1, c2_W2, c2_b2, c3_W1, c3_b1, c3_W2, c3_b2,
           head_W, head_b):
    src = edge_index[0]
    dst = edge_index[1]

    w1i_t = c1_W1[:, :_D].T
    w1j_t = c1_W1[:, _D:2 * _D].T
    w1e_t = c1_W1[:, 2 * _D:].T
    w2i_t = c2_W1[:, :_H].T
    w2j_t = c2_W1[:, _H:2 * _H].T
    w2e_t = c2_W1[:, 2 * _H:].T
    w3i_t = c3_W1[:, :_H].T
    w3j_t = c3_W1[:, _H:2 * _H].T
    w3e_t = c3_W1[:, 2 * _H:].T

    eids, dlocs, cnts = _bucket(dst)
    ep1, ep2, ep3 = _eproj_tc(edge_attr, w1e_t, w2e_t, w3e_t)

    a1, b1v = _proj_tc(x, w1i_t, w1j_t, c1_b1.reshape(1, _H))
    pre1 = _gather_pre(a1, b1v, ep1, src, dst)
    m1 = _mlp2_tc(pre1, c1_W2.T, c1_b2.reshape(1, _H))
    h1 = _seg_min(m1, eids, dlocs, cnts)

    a2, b2v = _proj_tc(h1, w2i_t, w2j_t, c2_b1.reshape(1, _H))
    pre2 = _gather_pre(a2, b2v, ep2, src, dst)
    m2 = _mlp2_tc(pre2, c2_W2.T, c2_b2.reshape(1, _H))
    h2 = _seg_min(m2, eids, dlocs, cnts)

    a3, b3v = _proj_tc(h2, w3i_t, w3j_t, c3_b1.reshape(1, _H))
    pre3 = _gather_pre(a3, b3v, ep3, src, dst)
    m3 = _mlp2_tc(pre3, c3_W2.T, c3_b2.reshape(1, _H))
    h3 = _seg_min(m3, eids, dlocs, cnts)

    alpha = _head_tc(x, h3[:_N], head_W[:, :_D].T, head_W[:, _D:].T,
                     head_b.reshape(1, 1))
    return alpha
